# Initial kernel scaffold; baseline (speedup 1.0000x reference)
import jax
import jax.numpy as jnp
from jax.experimental import pallas as pl


def kernel(image, image_ids, rois):
    raise NotImplementedError("write your pallas kernel here")


# speedup vs baseline: 138.9255x; 138.9255x over previous
import functools

import jax
import jax.numpy as jnp
from jax.experimental import pallas as pl
from jax.experimental.pallas import tpu as pltpu

POOLED = 7
RED = 16.0
NEG = float(jnp.finfo(jnp.float32).min)


def _roi_meta(rois, image_ids, H, W):
    R4 = rois.reshape(-1, 4)
    scale = jnp.float32(1.0 / RED)
    xs = jnp.round(R4[:, 0] * scale).astype(jnp.int32)
    ys = jnp.round(R4[:, 1] * scale).astype(jnp.int32)
    xe = jnp.round(R4[:, 2] * scale).astype(jnp.int32)
    ye = jnp.round(R4[:, 3] * scale).astype(jnp.int32)
    roi_w = jnp.maximum(xe - xs + 1, 1).astype(jnp.float32)
    roi_h = jnp.maximum(ye - ys + 1, 1).astype(jnp.float32)
    bin_h = roi_h / POOLED
    bin_w = roi_w / POOLED
    p = jnp.arange(POOLED, dtype=jnp.float32)
    hs = jnp.clip(jnp.floor(p[None, :] * bin_h[:, None]).astype(jnp.int32) + ys[:, None], 0, H)
    he = jnp.clip(jnp.ceil((p[None, :] + 1.0) * bin_h[:, None]).astype(jnp.int32) + ys[:, None], 0, H)
    ws = jnp.clip(jnp.floor(p[None, :] * bin_w[:, None]).astype(jnp.int32) + xs[:, None], 0, W)
    we = jnp.clip(jnp.ceil((p[None, :] + 1.0) * bin_w[:, None]).astype(jnp.int32) + xs[:, None], 0, W)
    hl = he - hs
    wl = we - ws
    nroi = R4.shape[0]
    nimg = image_ids.shape[0]
    per = nroi // nimg
    bimg = jnp.repeat(image_ids.astype(jnp.int32), per)
    pad = jnp.zeros((nroi, 1), jnp.int32)
    meta = jnp.concatenate(
        [hs, pad, hl, pad, ws, pad, wl, bimg[:, None]], axis=1
    )
    return meta


def _body(meta_ref, feat_ref, out_ref, rowacc_ref):
    r = pl.program_id(0)
    for ph in range(POOLED):
        hs = meta_ref[r, ph]
        hl = meta_ref[r, 8 + ph]

        def h_step(k, acc, hs=hs):
            return jnp.maximum(acc, feat_ref[0, hs + k, :, :])

        rowacc_ref[:, :] = jax.lax.fori_loop(
            0, hl, h_step, jnp.full((50, 256), NEG, jnp.float32)
        )
        for pw in range(POOLED):
            ws = meta_ref[r, 16 + pw]
            wl = meta_ref[r, 24 + pw]

            def w_step(k, acc, ws=ws):
                return jnp.maximum(acc, rowacc_ref[pl.ds(ws + k, 1), :])

            binacc = jax.lax.fori_loop(
                0, wl, w_step, jnp.full((1, 256), NEG, jnp.float32)
            )
            empty = jnp.logical_or(hl == 0, wl == 0)
            out_ref[0, ph, pw, :] = jnp.where(empty, 0.0, binacc[0])


@functools.partial(jax.jit, static_argnums=())
def _run(featT, meta):
    nroi = meta.shape[0]
    grid_spec = pltpu.PrefetchScalarGridSpec(
        num_scalar_prefetch=1,
        grid=(nroi,),
        in_specs=[
            pl.BlockSpec(
                (1, 50, 50, 256), lambda r, meta: (meta[r, 31], 0, 0, 0)
            ),
        ],
        out_specs=pl.BlockSpec((1, POOLED, POOLED, 256), lambda r, meta: (r, 0, 0, 0)),
        scratch_shapes=[pltpu.VMEM((50, 256), jnp.float32)],
    )
    out = pl.pallas_call(
        _body,
        grid_spec=grid_spec,
        out_shape=jax.ShapeDtypeStruct((nroi, POOLED, POOLED, 256), jnp.float32),
    )(meta, featT)
    return out


def kernel(image, image_ids, rois):
    B, C, H, W = image.shape
    featT = jnp.transpose(image, (0, 2, 3, 1))
    meta = _roi_meta(rois, image_ids, H, W)
    out = _run(featT, meta)
    return jnp.transpose(out, (0, 3, 1, 2))
